# baseline (speedup 1.0000x reference)
import functools

import jax
import jax.numpy as jnp
from jax import lax
from jax.experimental import pallas as pl
from jax.experimental.pallas import tpu as pltpu


def _bdot(a, b):
    return lax.dot_general(a.astype(jnp.bfloat16), b.astype(jnp.bfloat16),
                           (((0,), (0,)), ((), ())),
                           preferred_element_type=jnp.float32)


def _cdot_high(a, b):
    return lax.dot_general(a, b, (((1,), (0,)), ((), ())),
                           preferred_element_type=jnp.float32,
                           precision=lax.Precision.HIGHEST)


def _fe_body(nblocks, pts_ref, w_in_ref, b_in_ref, w_blk_ref, b_blk_ref,
             feats_ref, gmax_ref, psq_ref):
    j = pl.program_id(1)
    x = pts_ref[0]
    psq_ref[0] = jnp.sum(x * x, axis=0, keepdims=True)
    f = jnp.maximum(_bdot(w_in_ref[...], x) + b_in_ref[...], 0.0)
    fs = [f]
    for i in range(nblocks):
        f = jnp.maximum(_bdot(w_blk_ref[i], f) + b_blk_ref[i], 0.0)
        fs.append(f)
    feats_ref[0] = jnp.concatenate(fs, axis=0).astype(jnp.bfloat16)
    m = jnp.max(f, axis=1, keepdims=True)

    @pl.when(j == 0)
    def _():
        gmax_ref[0] = m

    @pl.when(j != 0)
    def _():
        gmax_ref[0] = jnp.maximum(gmax_ref[0], m)


def _knn_body(n, k, pts_ref, q_ref, psq_ref, iw_ref):
    p = pts_ref[0]
    q = q_ref[0]
    psq = psq_ref[0]
    qsq = jnp.sum(q * q, axis=0, keepdims=True)
    pq = _bdot(p, q)
    d2 = (qsq - 2.0 * pq) + psq
    iota_col = lax.broadcasted_iota(jnp.int32, (n, 1), 0).astype(jnp.float32)

    rows = []
    vals = []
    for _ in range(k):
        v = jnp.min(d2, axis=0, keepdims=True)
        m = d2 == v
        ind = jnp.where(m, 1.0, 0.0)
        i = lax.dot_general(iota_col, ind, (((0,), (0,)), ((), ())),
                            preferred_element_type=jnp.float32,
                            precision=lax.Precision.HIGHEST)
        d2 = jnp.where(m, jnp.inf, d2)
        rows.append(i)
        vals.append(v)

    ws = [1.0 / (jnp.maximum(v, 0.0) + 1e-8) for v in vals]
    wsum = ws[0] + ws[1] + ws[2]
    ws = [w / wsum for w in ws]
    iw_ref[0] = jnp.concatenate(rows + ws, axis=0)


def _reg_body(nc_blocks, nc, k, feats_ref, iw_ref, q_ref, gmax_ref,
              wr1q_ref, wr1l_ref, wr1g_ref, wr2_ref, wr3_ref,
              br1_ref, br2_ref, br3_ref, out_ref, acc_ref):
    c = pl.program_id(2)
    iw = iw_ref[0]
    mb = iw.shape[1]
    iota = lax.broadcasted_iota(jnp.int32, (nc, mb), 0) + c * nc
    wmat_t = jnp.zeros((nc, mb), jnp.float32)
    for kk in range(k):
        idx = iw[kk:kk + 1, :].astype(jnp.int32)
        w = iw[k + kk:k + kk + 1, :]
        wmat_t = wmat_t + jnp.where(iota == idx, w, 0.0)
    part = lax.dot_general(feats_ref[0], wmat_t.astype(jnp.bfloat16),
                           (((1,), (0,)), ((), ())),
                           preferred_element_type=jnp.float32)

    @pl.when(c == 0)
    def _():
        acc_ref[...] = part

    @pl.when(c != 0)
    def _():
        acc_ref[...] = acc_ref[...] + part

    @pl.when(c == nc_blocks - 1)
    def _():
        interp = acc_ref[...]
        h1 = (_bdot(wr1l_ref[...], interp)
              + _bdot(wr1q_ref[...], q_ref[0])
              + _bdot(wr1g_ref[...], gmax_ref[0])
              + br1_ref[...])
        h1 = jnp.maximum(h1, 0.0)
        h2 = jnp.maximum(_bdot(wr2_ref[...], h1) + br2_ref[...], 0.0)
        out_ref[0] = _bdot(wr3_ref[...], h2) + br3_ref[...]


def _full(shape):
    nd = len(shape)
    return pl.BlockSpec(shape, lambda *_: (0,) * nd)


@jax.jit
def kernel(original_pts, query_pts, W_in, b_in, W_blk, b_blk,
           Wr1, br1, Wr2, br2, Wr3, br3):
    B, _, N = original_pts.shape
    M = query_pts.shape[2]
    C = W_in.shape[1]
    BN = W_blk.shape[0]
    H = Wr1.shape[1]
    K = 3
    C4 = (BN + 1) * C

    NB = 1024
    MB = 512
    MB3 = 512
    NC = 1024

    feats, gmax, psq_row = pl.pallas_call(
        functools.partial(_fe_body, BN),
        grid=(B, N // NB),
        in_specs=[
            pl.BlockSpec((1, 3, NB), lambda b, j: (b, 0, j)),
            _full((3, C)),
            _full((C, 1)),
            _full((BN, C, C)),
            _full((BN, C, 1)),
        ],
        out_specs=[
            pl.BlockSpec((1, C4, NB), lambda b, j: (b, 0, j)),
            pl.BlockSpec((1, C, 1), lambda b, j: (b, 0, 0)),
            pl.BlockSpec((1, 1, NB), lambda b, j: (b, 0, j)),
        ],
        out_shape=[
            jax.ShapeDtypeStruct((B, C4, N), jnp.bfloat16),
            jax.ShapeDtypeStruct((B, C, 1), jnp.float32),
            jax.ShapeDtypeStruct((B, 1, N), jnp.float32),
        ],
    )(original_pts, W_in, b_in.reshape(C, 1), W_blk, b_blk.reshape(BN, C, 1))

    psq_col = psq_row.reshape(B, N, 1)

    iw = pl.pallas_call(
        functools.partial(_knn_body, N, K),
        grid=(B, M // MB),
        in_specs=[
            pl.BlockSpec((1, 3, N), lambda b, j: (b, 0, 0)),
            pl.BlockSpec((1, 3, MB), lambda b, j: (b, 0, j)),
            pl.BlockSpec((1, N, 1), lambda b, j: (b, 0, 0)),
        ],
        out_specs=pl.BlockSpec((1, 2 * K, MB), lambda b, j: (b, 0, j)),
        out_shape=jax.ShapeDtypeStruct((B, 2 * K, M), jnp.float32),
    )(original_pts, query_pts, psq_col)

    Wr1q = Wr1[:3]
    Wr1l = Wr1[3:3 + C4]
    Wr1g = Wr1[3 + C4:]

    out = pl.pallas_call(
        functools.partial(_reg_body, N // NC, NC, K),
        grid=(B, M // MB3, N // NC),
        in_specs=[
            pl.BlockSpec((1, C4, NC), lambda b, j, c: (b, 0, c)),
            pl.BlockSpec((1, 2 * K, MB3), lambda b, j, c: (b, 0, j)),
            pl.BlockSpec((1, 3, MB3), lambda b, j, c: (b, 0, j)),
            pl.BlockSpec((1, C, 1), lambda b, j, c: (b, 0, 0)),
            _full((3, H)),
            _full((C4, H)),
            _full((C, H)),
            _full((H, H)),
            _full((H, 1)),
            _full((H, 1)),
            _full((H, 1)),
            _full((1, 1)),
        ],
        out_specs=pl.BlockSpec((1, 1, MB3), lambda b, j, c: (b, 0, j)),
        out_shape=jax.ShapeDtypeStruct((B, 1, M), jnp.float32),
        scratch_shapes=[pltpu.VMEM((C4, MB3), jnp.float32)],
    )(feats, iw, query_pts, gmax, Wr1q, Wr1l, Wr1g, Wr2, Wr3,
      br1.reshape(H, 1), br2.reshape(H, 1), br3.reshape(1, 1))

    return out


# speedup vs baseline: 1.5046x; 1.5046x over previous
import functools

import jax
import jax.numpy as jnp
from jax import lax
from jax.experimental import pallas as pl
from jax.experimental.pallas import tpu as pltpu


def _bdot(a, b):
    return lax.dot_general(a.astype(jnp.bfloat16), b.astype(jnp.bfloat16),
                           (((0,), (0,)), ((), ())),
                           preferred_element_type=jnp.float32)


def _cdot_high(a, b):
    return lax.dot_general(a, b, (((1,), (0,)), ((), ())),
                           preferred_element_type=jnp.float32,
                           precision=lax.Precision.HIGHEST)


def _fe_body(nblocks, pts_ref, w_in_ref, b_in_ref, w_blk_ref, b_blk_ref,
             feats_ref, gmax_ref, psq_ref):
    j = pl.program_id(1)
    x = pts_ref[0]
    psq_ref[0] = jnp.sum(x * x, axis=0, keepdims=True)
    f = jnp.maximum(_bdot(w_in_ref[...], x) + b_in_ref[...], 0.0)
    fs = [f]
    for i in range(nblocks):
        f = jnp.maximum(_bdot(w_blk_ref[i], f) + b_blk_ref[i], 0.0)
        fs.append(f)
    feats_ref[0] = jnp.concatenate(fs, axis=0).astype(jnp.bfloat16)
    m = jnp.max(f, axis=1, keepdims=True)

    @pl.when(j == 0)
    def _():
        gmax_ref[0] = m

    @pl.when(j != 0)
    def _():
        gmax_ref[0] = jnp.maximum(gmax_ref[0], m)


def _knn_body(n, k, pts_ref, q_ref, psq_ref, iw_ref):
    p = pts_ref[0]
    q = q_ref[0]
    psq = psq_ref[0]
    qsq = jnp.sum(q * q, axis=0, keepdims=True)
    pq = _bdot(p, q)
    d2 = (qsq - 2.0 * pq) + psq
    iota = lax.broadcasted_iota(jnp.int32, d2.shape, 0)

    rows = []
    vals = []
    for _ in range(k):
        v = jnp.min(d2, axis=0, keepdims=True)
        m = d2 == v
        i = jnp.min(jnp.where(m, iota, n), axis=0, keepdims=True)
        d2 = jnp.where(m, jnp.inf, d2)
        rows.append(i.astype(jnp.float32))
        vals.append(v)

    ws = [1.0 / (jnp.maximum(v, 0.0) + 1e-8) for v in vals]
    wsum = ws[0] + ws[1] + ws[2]
    ws = [w / wsum for w in ws]
    iw_ref[0] = jnp.concatenate(rows + ws, axis=0)


def _reg_body(nc_blocks, nc, k, feats_ref, iw_ref, q_ref, gmax_ref,
              wr1q_ref, wr1l_ref, wr1g_ref, wr2_ref, wr3_ref,
              br1_ref, br2_ref, br3_ref, out_ref, acc_ref):
    c = pl.program_id(2)
    iw = iw_ref[0]
    mb = iw.shape[1]
    iota = lax.broadcasted_iota(jnp.int32, (nc, mb), 0) + c * nc
    wmat_t = jnp.zeros((nc, mb), jnp.float32)
    for kk in range(k):
        idx = iw[kk:kk + 1, :].astype(jnp.int32)
        w = iw[k + kk:k + kk + 1, :]
        wmat_t = wmat_t + jnp.where(iota == idx, w, 0.0)
    part = lax.dot_general(feats_ref[0], wmat_t.astype(jnp.bfloat16),
                           (((1,), (0,)), ((), ())),
                           preferred_element_type=jnp.float32)

    @pl.when(c == 0)
    def _():
        acc_ref[...] = part

    @pl.when(c != 0)
    def _():
        acc_ref[...] = acc_ref[...] + part

    @pl.when(c == nc_blocks - 1)
    def _():
        interp = acc_ref[...]
        h1 = (_bdot(wr1l_ref[...], interp)
              + _bdot(wr1q_ref[...], q_ref[0])
              + _bdot(wr1g_ref[...], gmax_ref[0])
              + br1_ref[...])
        h1 = jnp.maximum(h1, 0.0)
        h2 = jnp.maximum(_bdot(wr2_ref[...], h1) + br2_ref[...], 0.0)
        out_ref[0] = _bdot(wr3_ref[...], h2) + br3_ref[...]


def _full(shape):
    nd = len(shape)
    return pl.BlockSpec(shape, lambda *_: (0,) * nd)


@jax.jit
def kernel(original_pts, query_pts, W_in, b_in, W_blk, b_blk,
           Wr1, br1, Wr2, br2, Wr3, br3):
    B, _, N = original_pts.shape
    M = query_pts.shape[2]
    C = W_in.shape[1]
    BN = W_blk.shape[0]
    H = Wr1.shape[1]
    K = 3
    C4 = (BN + 1) * C

    NB = 1024
    MB = 512
    MB3 = 512
    NC = 1024

    feats, gmax, psq_row = pl.pallas_call(
        functools.partial(_fe_body, BN),
        grid=(B, N // NB),
        in_specs=[
            pl.BlockSpec((1, 3, NB), lambda b, j: (b, 0, j)),
            _full((3, C)),
            _full((C, 1)),
            _full((BN, C, C)),
            _full((BN, C, 1)),
        ],
        out_specs=[
            pl.BlockSpec((1, C4, NB), lambda b, j: (b, 0, j)),
            pl.BlockSpec((1, C, 1), lambda b, j: (b, 0, 0)),
            pl.BlockSpec((1, 1, NB), lambda b, j: (b, 0, j)),
        ],
        out_shape=[
            jax.ShapeDtypeStruct((B, C4, N), jnp.bfloat16),
            jax.ShapeDtypeStruct((B, C, 1), jnp.float32),
            jax.ShapeDtypeStruct((B, 1, N), jnp.float32),
        ],
    )(original_pts, W_in, b_in.reshape(C, 1), W_blk, b_blk.reshape(BN, C, 1))

    psq_col = psq_row.reshape(B, N, 1)

    iw = pl.pallas_call(
        functools.partial(_knn_body, N, K),
        grid=(B, M // MB),
        in_specs=[
            pl.BlockSpec((1, 3, N), lambda b, j: (b, 0, 0)),
            pl.BlockSpec((1, 3, MB), lambda b, j: (b, 0, j)),
            pl.BlockSpec((1, N, 1), lambda b, j: (b, 0, 0)),
        ],
        out_specs=pl.BlockSpec((1, 2 * K, MB), lambda b, j: (b, 0, j)),
        out_shape=jax.ShapeDtypeStruct((B, 2 * K, M), jnp.float32),
    )(original_pts, query_pts, psq_col)

    Wr1q = Wr1[:3]
    Wr1l = Wr1[3:3 + C4]
    Wr1g = Wr1[3 + C4:]

    out = pl.pallas_call(
        functools.partial(_reg_body, N // NC, NC, K),
        grid=(B, M // MB3, N // NC),
        in_specs=[
            pl.BlockSpec((1, C4, NC), lambda b, j, c: (b, 0, c)),
            pl.BlockSpec((1, 2 * K, MB3), lambda b, j, c: (b, 0, j)),
            pl.BlockSpec((1, 3, MB3), lambda b, j, c: (b, 0, j)),
            pl.BlockSpec((1, C, 1), lambda b, j, c: (b, 0, 0)),
            _full((3, H)),
            _full((C4, H)),
            _full((C, H)),
            _full((H, H)),
            _full((H, 1)),
            _full((H, 1)),
            _full((H, 1)),
            _full((1, 1)),
        ],
        out_specs=pl.BlockSpec((1, 1, MB3), lambda b, j, c: (b, 0, j)),
        out_shape=jax.ShapeDtypeStruct((B, 1, M), jnp.float32),
        scratch_shapes=[pltpu.VMEM((C4, MB3), jnp.float32)],
    )(feats, iw, query_pts, gmax, Wr1q, Wr1l, Wr1g, Wr2, Wr3,
      br1.reshape(H, 1), br2.reshape(H, 1), br3.reshape(1, 1))

    return out
